# baseline (speedup 1.0000x reference)
import functools

import jax
import jax.numpy as jnp
from jax import lax
from jax.experimental import pallas as pl
from jax.experimental.pallas import tpu as pltpu
from jax.experimental.pallas import tpu_sc as plsc

B, H, W, C = 8, 224, 224, 4
N = H * W
NV = N // 16
NC = 2
NS = 16
NL = NV + 1
CHL = 512
SROWS = 3584


def _pack_body(x_ref, out_ref):
    def q(v):
        return jnp.clip((v + 1.0) * 127.5, 0.0, 255.0).astype(jnp.int32)

    code = q(x_ref[0, 0, :])
    for j in (1, 2, 3):
        code = lax.shift_left(code, 8) | q(x_ref[0, j, :])
    out_ref[0, 0, :] = code


_sc_mesh = plsc.VectorSubcoreMesh(core_axis_name="c", subcore_axis_name="s")


@functools.partial(
    pl.kernel,
    out_type=(
        jax.ShapeDtypeStruct((B * NL, 16), jnp.int32),
        jax.ShapeDtypeStruct((B, 16), jnp.int32),
    ),
    mesh=_sc_mesh,
    compiler_params=pltpu.CompilerParams(needs_layout_passes=False, use_tc_tiling_on_sc=False),
    scratch_types=[
        pltpu.VMEM((SROWS, 16), jnp.int32),
        pltpu.VMEM((N,), jnp.int32),
        pltpu.VMEM((4096,), jnp.int32),
        pltpu.VMEM((16,), jnp.int32),
        pltpu.VMEM((64,), jnp.int32),
        pltpu.VMEM((16,), jnp.int32),
        pltpu.VMEM((CHL,), jnp.int32),
        pltpu.HBM((2, 16, 16), jnp.int32),
        pltpu.SemaphoreType.DMA,
    ],
)
def _sc_sort(codes_hbm, out_hbm, cnt_hbm, stg, buf_b, hist, cvec, hbuf, sbuf,
             idxl, shared, sem):
    cid = lax.axis_index("c")
    sid = lax.axis_index("s")
    g = sid // 4
    t = sid % 4
    b = cid * 4 + g
    iota = lax.iota(jnp.int32, 16)

    pltpu.sync_copy(codes_hbm.at[b], stg.at[pl.ds(0, NV), :])

    def comp(i, pos):
        v = stg[i, :]
        m = lax.shift_right_logical(v, jnp.full((16,), 30, jnp.int32)) == t
        plsc.store_compressed(buf_b.at[pl.ds(pos, 16)], v, mask=m)
        return pos + plsc.all_reduce_population_count(m)[0]

    nt = plsc.parallel_loop(0, NV, 1, unroll=4, carry=jnp.int32(0))(comp)
    nvt = (nt + 15) // 16

    hq = (nvt + 3) // 4

    def radix_pass(read_vreg, scatter_dst, sh):
        shv = jnp.full((16,), sh, jnp.int32)

        def zero(j):
            hist[pl.ds(j * 16, 16)] = jnp.zeros((16,), jnp.int32)

        plsc.parallel_loop(0, 256, 1, unroll=4)(zero)

        def chunk_vals(i, c):
            vr = c * hq + i
            v = read_vreg(jnp.minimum(vr, NV - 1))
            valid = iota + vr * 16 < nt
            d = (lax.shift_right_logical(v, shv) & 1023) + c * 1024
            rank, last = plsc.scan_count(d, mask=valid)
            return v, valid, d, rank, last

        def hphase(i):
            for c in range(4):
                v, valid, d, rank, last = chunk_vals(i, c)
                plsc.addupdate_scatter(hist, [d], rank, mask=last)

        plsc.parallel_loop(0, hq, 1, unroll=2)(hphase)

        def sphase(j, carry):
            hs = [hist[pl.ds(c * 1024 + j * 16, 16)] for c in range(4)]
            tot = hs[0] + hs[1] + hs[2] + hs[3]
            incl = plsc.cumsum(tot)
            base = incl - tot + carry
            for c in range(4):
                hist[pl.ds(c * 1024 + j * 16, 16)] = base
                base = base + hs[c]
            return carry + jnp.sum(tot)

        lax.fori_loop(0, 64, sphase, jnp.int32(0))

        def pphase(i, _):
            for c in range(4):
                v, valid, d, rank, last = chunk_vals(i, c)
                base = plsc.load_gather(hist, [d])
                scatter_dst(base + rank - 1, v, valid)
                plsc.addupdate_scatter(hist, [d], rank, mask=last)
            return 0

        lax.fori_loop(0, hq, pphase, 0)

    def read_b(i):
        return buf_b[pl.ds(i * 16, 16)]

    def read_stg(i):
        return stg[i, :]

    def scat_stg(dest, v, m):
        plsc.store_scatter(stg, [lax.shift_right_logical(dest, 4), dest & 15],
                           v, mask=m)

    def scat_b(dest, v, m):
        plsc.store_scatter(buf_b, [dest], v, mask=m)

    radix_pass(read_b, scat_stg, 0)
    radix_pass(read_stg, scat_b, 10)
    radix_pass(read_b, scat_stg, 20)

    def dphase(i, pos):
        v = stg[i, :]
        idxv = iota + i * 16
        pidx = jnp.maximum(idxv - 1, 0)
        pv = plsc.load_gather(stg, [lax.shift_right_logical(pidx, 4),
                                    pidx & 15])
        m = ((v != pv) | (idxv == 0)) & (idxv < nt)
        plsc.store_compressed(buf_b.at[pl.ds(pos, 16)], v, mask=m)
        return pos + plsc.all_reduce_population_count(m)[0]

    ut = plsc.parallel_loop(0, nvt, 1, unroll=4, carry=jnp.int32(0))(dphase)

    hv = plsc.load_gather(buf_b, [jnp.maximum(iota - 1, 0)])
    cvec[...] = jnp.where(iota == 0, ut, hv)
    pltpu.sync_copy(cvec, shared.at[cid, sid])
    plsc.subcore_barrier()
    us = []
    for tt in range(4):
        pltpu.sync_copy(shared.at[cid, g * 4 + tt], hbuf.at[pl.ds(tt * 16, 16)])
        row = hbuf[pl.ds(tt * 16, 16)]
        us.append(jnp.sum(jnp.where(iota == 0, row, 0)))
    off = (jnp.where(t > 0, us[0], 0) + jnp.where(t > 1, us[1], 0)
           + jnp.where(t > 2, us[2], 0))
    cnt = us[0] + us[1] + us[2] + us[3]

    @pl.when(t == 0)
    def _():
        cvec[...] = jnp.full((16,), cnt, jnp.int32)
        pltpu.sync_copy(cvec, cnt_hbm.at[b])

    sv = jnp.zeros((16,), jnp.int32)
    pos = jnp.int32(0)
    for tt in range(1, 4):
        act = tt > t
        val = plsc.load_gather(
            hbuf, [tt * 16 + 1 + jnp.clip(iota - pos, 0, 14)])
        m = act & (iota >= pos) & (iota < pos + us[tt])
        sv = jnp.where(m, val, sv)
        pos = pos + jnp.where(act, us[tt], 0)
    sbuf[...] = sv

    l0 = (off + 15) >> 4
    nlines = ((off + ut + 15) >> 4) - l0
    h16 = l0 * 16 - off

    def stage(k):
        gidx = h16 + k * 16 + iota
        gb = plsc.load_gather(buf_b, [jnp.clip(gidx, 0, N - 1)])
        gs = plsc.load_gather(sbuf, [jnp.clip(gidx - ut, 0, 15)])
        stg[k, :] = jnp.where(gidx < ut, gb, gs)

    plsc.parallel_loop(0, nlines, 1, unroll=4)(stage)

    lbase = b * NL + l0
    dummy = b * NL + NV

    def outchunk(ci, _):
        for j in range(CHL // 16):
            jj = iota + ci * CHL + j * 16
            gi = jnp.where(jj < nlines, lbase + jj, dummy)
            idxl[pl.ds(j * 16, 16)] = gi
        pltpu.async_copy(stg.at[pl.ds(ci * CHL, CHL), :], out_hbm.at[idxl],
                         sem).wait()
        return 0

    lax.fori_loop(0, (nlines + CHL - 1) // CHL, outchunk, 0)


def _decode_body(codes_ref, counts_ref, out_ref):
    c = codes_ref[0, 0, :]
    idx = lax.broadcasted_iota(jnp.int32, (N,), 0)
    c = jnp.where(idx < counts_ref[0, 0, 0], c, jnp.int32(0))
    for j, sh in enumerate((24, 16, 8, 0)):
        ch = lax.shift_right_logical(c, jnp.int32(sh)) & 255
        out_ref[0, j, :] = ch.astype(jnp.float32) * (1.0 / 127.5) - 1.0


def kernel(images):
    xt = images.reshape(B, N, 4).transpose(0, 2, 1)
    codes = pl.pallas_call(
        _pack_body,
        grid=(B,),
        in_specs=[pl.BlockSpec((1, 4, N), lambda b: (b, 0, 0))],
        out_specs=pl.BlockSpec((1, 1, N), lambda b: (b, 0, 0)),
        out_shape=jax.ShapeDtypeStruct((B, 1, N), jnp.int32),
    )(xt)

    out_lines, cnt16 = _sc_sort(codes.reshape(B, NV, 16))
    counts = cnt16[:, 0]
    sorted_codes = out_lines.reshape(B, NL * 16)[:, :N]

    out = pl.pallas_call(
        _decode_body,
        grid=(B,),
        in_specs=[
            pl.BlockSpec((1, 1, N), lambda b: (b, 0, 0)),
            pl.BlockSpec((1, 1, 1), lambda b: (b, 0, 0), memory_space=pltpu.SMEM),
        ],
        out_specs=pl.BlockSpec((1, 4, N), lambda b: (b, 0, 0)),
        out_shape=jax.ShapeDtypeStruct((B, 4, N), jnp.float32),
    )(sorted_codes.reshape(B, 1, N), counts.reshape(B, 1, 1))
    palettes = out.transpose(0, 2, 1)
    return palettes, counts


# speedup vs baseline: 1.0659x; 1.0659x over previous
import functools

import jax
import jax.numpy as jnp
from jax import lax
from jax.experimental import pallas as pl
from jax.experimental.pallas import tpu as pltpu
from jax.experimental.pallas import tpu_sc as plsc

B, H, W, C = 8, 224, 224, 4
N = H * W
NV = N // 16
NC = 2
NS = 16
NL = NV + 1
CHL = 512
SROWS = 3584


def _pack_body(x_ref, out_ref):
    def q(v):
        return jnp.clip((v + 1.0) * 127.5, 0.0, 255.0).astype(jnp.int32)

    code = q(x_ref[0, 0, :])
    for j in (1, 2, 3):
        code = lax.shift_left(code, 8) | q(x_ref[0, j, :])
    out_ref[0, 0, :] = code


_sc_mesh = plsc.VectorSubcoreMesh(core_axis_name="c", subcore_axis_name="s")


@functools.partial(
    pl.kernel,
    out_type=(
        jax.ShapeDtypeStruct((B * NL, 16), jnp.int32),
        jax.ShapeDtypeStruct((B, 16), jnp.int32),
    ),
    mesh=_sc_mesh,
    compiler_params=pltpu.CompilerParams(needs_layout_passes=False, use_tc_tiling_on_sc=False),
    scratch_types=[
        pltpu.VMEM((SROWS, 16), jnp.int32),
        pltpu.VMEM((N,), jnp.int32),
        pltpu.VMEM((1024,), jnp.int32),
        pltpu.VMEM((16,), jnp.int32),
        pltpu.VMEM((64,), jnp.int32),
        pltpu.VMEM((16,), jnp.int32),
        pltpu.VMEM((CHL,), jnp.int32),
        pltpu.HBM((2, 16, 16), jnp.int32),
        pltpu.SemaphoreType.DMA,
    ],
)
def _sc_sort(codes_hbm, out_hbm, cnt_hbm, stg, buf_b, hist, cvec, hbuf, sbuf,
             idxl, shared, sem):
    cid = lax.axis_index("c")
    sid = lax.axis_index("s")
    g = sid // 4
    t = sid % 4
    b = cid * 4 + g
    iota = lax.iota(jnp.int32, 16)

    pltpu.sync_copy(codes_hbm.at[b], stg.at[pl.ds(0, NV), :])

    def comp(i, pos):
        v = stg[i, :]
        m = lax.shift_right_logical(v, jnp.full((16,), 30, jnp.int32)) == t
        plsc.store_compressed(buf_b.at[pl.ds(pos, 16)], v, mask=m)
        return pos + plsc.all_reduce_population_count(m)[0]

    nt = plsc.parallel_loop(0, NV, 1, unroll=4, carry=jnp.int32(0))(comp)
    nvt = (nt + 15) // 16

    ones = jnp.ones((16,), jnp.int32)

    def radix_pass(read_vreg, scatter_dst, sh):
        shv = jnp.full((16,), sh, jnp.int32)

        def zero(j):
            hist[pl.ds(j * 16, 16)] = jnp.zeros((16,), jnp.int32)

        plsc.parallel_loop(0, 64, 1, unroll=4)(zero)

        def hphase(i):
            v = read_vreg(i)
            valid = iota + i * 16 < nt
            d = lax.shift_right_logical(v, shv) & 1023
            plsc.addupdate_scatter(hist, [d], ones, mask=valid)

        plsc.parallel_loop(0, nvt, 1, unroll=4)(hphase)

        def sphase(j, carry):
            h = hist[pl.ds(j * 16, 16)]
            incl = plsc.cumsum(h)
            hist[pl.ds(j * 16, 16)] = incl - h + carry
            return carry + jnp.sum(h)

        lax.fori_loop(0, 64, sphase, jnp.int32(0))

        def pphase(i, _):
            v = read_vreg(i)
            valid = iota + i * 16 < nt
            d = lax.shift_right_logical(v, shv) & 1023
            rank, last = plsc.scan_count(d, mask=valid)
            base = plsc.load_gather(hist, [d])
            scatter_dst(base + rank - 1, v, valid)
            plsc.addupdate_scatter(hist, [d], rank, mask=last)
            return 0

        lax.fori_loop(0, nvt, pphase, 0)

    def read_b(i):
        return buf_b[pl.ds(i * 16, 16)]

    def read_stg(i):
        return stg[i, :]

    def scat_stg(dest, v, m):
        plsc.store_scatter(stg, [lax.shift_right_logical(dest, 4), dest & 15],
                           v, mask=m)

    def scat_b(dest, v, m):
        plsc.store_scatter(buf_b, [dest], v, mask=m)

    radix_pass(read_b, scat_stg, 0)
    radix_pass(read_stg, scat_b, 10)
    radix_pass(read_b, scat_stg, 20)

    def dphase(i, pos):
        v = stg[i, :]
        idxv = iota + i * 16
        pidx = jnp.maximum(idxv - 1, 0)
        pv = plsc.load_gather(stg, [lax.shift_right_logical(pidx, 4),
                                    pidx & 15])
        m = ((v != pv) | (idxv == 0)) & (idxv < nt)
        plsc.store_compressed(buf_b.at[pl.ds(pos, 16)], v, mask=m)
        return pos + plsc.all_reduce_population_count(m)[0]

    ut = plsc.parallel_loop(0, nvt, 1, unroll=4, carry=jnp.int32(0))(dphase)

    hv = plsc.load_gather(buf_b, [jnp.maximum(iota - 1, 0)])
    cvec[...] = jnp.where(iota == 0, ut, hv)
    pltpu.sync_copy(cvec, shared.at[cid, sid])
    plsc.subcore_barrier()
    us = []
    for tt in range(4):
        pltpu.sync_copy(shared.at[cid, g * 4 + tt], hbuf.at[pl.ds(tt * 16, 16)])
        row = hbuf[pl.ds(tt * 16, 16)]
        us.append(jnp.sum(jnp.where(iota == 0, row, 0)))
    off = (jnp.where(t > 0, us[0], 0) + jnp.where(t > 1, us[1], 0)
           + jnp.where(t > 2, us[2], 0))
    cnt = us[0] + us[1] + us[2] + us[3]

    @pl.when(t == 0)
    def _():
        cvec[...] = jnp.full((16,), cnt, jnp.int32)
        pltpu.sync_copy(cvec, cnt_hbm.at[b])

    sv = jnp.zeros((16,), jnp.int32)
    pos = jnp.int32(0)
    for tt in range(1, 4):
        act = tt > t
        val = plsc.load_gather(
            hbuf, [tt * 16 + 1 + jnp.clip(iota - pos, 0, 14)])
        m = act & (iota >= pos) & (iota < pos + us[tt])
        sv = jnp.where(m, val, sv)
        pos = pos + jnp.where(act, us[tt], 0)
    sbuf[...] = sv

    l0 = (off + 15) >> 4
    nlines = ((off + ut + 15) >> 4) - l0
    h16 = l0 * 16 - off

    def stage(k):
        gidx = h16 + k * 16 + iota
        gb = plsc.load_gather(buf_b, [jnp.clip(gidx, 0, N - 1)])
        gs = plsc.load_gather(sbuf, [jnp.clip(gidx - ut, 0, 15)])
        stg[k, :] = jnp.where(gidx < ut, gb, gs)

    plsc.parallel_loop(0, nlines, 1, unroll=4)(stage)

    lbase = b * NL + l0
    dummy = b * NL + NV

    def outchunk(ci, _):
        for j in range(CHL // 16):
            jj = iota + ci * CHL + j * 16
            gi = jnp.where(jj < nlines, lbase + jj, dummy)
            idxl[pl.ds(j * 16, 16)] = gi
        pltpu.async_copy(stg.at[pl.ds(ci * CHL, CHL), :], out_hbm.at[idxl],
                         sem).wait()
        return 0

    lax.fori_loop(0, (nlines + CHL - 1) // CHL, outchunk, 0)


def _decode_body(codes_ref, counts_ref, out_ref):
    c = codes_ref[0, 0, :]
    idx = lax.broadcasted_iota(jnp.int32, (N,), 0)
    c = jnp.where(idx < counts_ref[0, 0, 0], c, jnp.int32(0))
    for j, sh in enumerate((24, 16, 8, 0)):
        ch = lax.shift_right_logical(c, jnp.int32(sh)) & 255
        out_ref[0, j, :] = ch.astype(jnp.float32) * (1.0 / 127.5) - 1.0


def kernel(images):
    xt = images.reshape(B, N, 4).transpose(0, 2, 1)
    codes = pl.pallas_call(
        _pack_body,
        grid=(B,),
        in_specs=[pl.BlockSpec((1, 4, N), lambda b: (b, 0, 0))],
        out_specs=pl.BlockSpec((1, 1, N), lambda b: (b, 0, 0)),
        out_shape=jax.ShapeDtypeStruct((B, 1, N), jnp.int32),
    )(xt)

    out_lines, cnt16 = _sc_sort(codes.reshape(B, NV, 16))
    counts = cnt16[:, 0]
    sorted_codes = out_lines.reshape(B, NL * 16)[:, :N]

    out = pl.pallas_call(
        _decode_body,
        grid=(B,),
        in_specs=[
            pl.BlockSpec((1, 1, N), lambda b: (b, 0, 0)),
            pl.BlockSpec((1, 1, 1), lambda b: (b, 0, 0), memory_space=pltpu.SMEM),
        ],
        out_specs=pl.BlockSpec((1, 4, N), lambda b: (b, 0, 0)),
        out_shape=jax.ShapeDtypeStruct((B, 4, N), jnp.float32),
    )(sorted_codes.reshape(B, 1, N), counts.reshape(B, 1, 1))
    palettes = out.transpose(0, 2, 1)
    return palettes, counts
